# Initial kernel scaffold; baseline (speedup 1.0000x reference)
import jax
import jax.numpy as jnp
from jax.experimental import pallas as pl


def kernel(feature_miRNA, feature_gene, feature_drug, mp_ins):
    raise NotImplementedError("write your pallas kernel here")


# speedup vs baseline: 6.1969x; 6.1969x over previous
import functools

import jax
import jax.numpy as jnp
from jax import lax
from jax.experimental import pallas as pl
from jax.experimental.pallas import tpu as pltpu
from jax.experimental.pallas import tpu_sc as plsc

NC = 2
NS = 16
NW = NC * NS
LANES = 16
W = 128


def _aggregate(mi_hbm, ge_hbm, dr_hbm, idx_hbm, out_hbm,
               idx_v, g0, g1, g2, g3, sem):
    wid = lax.axis_index("s") * NC + lax.axis_index("c")
    wpw = idx_hbm.shape[2]
    d = mi_hbm.shape[1]
    pltpu.sync_copy(idx_hbm.at[wid], idx_v)

    @pl.loop(0, wpw)
    def _(w):
        c0 = pltpu.async_copy(mi_hbm.at[idx_v.at[0, w]], g0, sem)
        c1 = pltpu.async_copy(ge_hbm.at[idx_v.at[1, w]], g1, sem)
        c2 = pltpu.async_copy(ge_hbm.at[idx_v.at[2, w]], g2, sem)
        c3 = pltpu.async_copy(dr_hbm.at[idx_v.at[3, w]], g3, sem)
        c0.wait()
        c1.wait()
        c2.wait()
        c3.wait()

        @pl.loop(0, W)
        def _(r):
            for c in range(0, d, LANES):
                s = (r, pl.ds(c, LANES))
                g0.at[s][...] = (g0.at[s][...] + g1.at[s][...]
                                 + g2.at[s][...] + g3.at[s][...])

        base = wid * (wpw * W) + w * W
        pltpu.sync_copy(g0, out_hbm.at[pl.ds(base, W)])


def kernel(feature_miRNA, feature_gene, feature_drug, mp_ins):
    b, l, p = mp_ins.shape
    v, d = feature_miRNA.shape
    n = b * l
    assert p == 4 and d % LANES == 0 and n % (NW * W) == 0
    wpw = n // (NW * W)

    idx = mp_ins.astype(jnp.int32).reshape(NW, wpw, W, 4).transpose(0, 3, 1, 2)

    mesh = plsc.VectorSubcoreMesh(core_axis_name="c", subcore_axis_name="s")
    run = pl.kernel(
        _aggregate,
        out_type=jax.ShapeDtypeStruct((n, d), jnp.float32),
        mesh=mesh,
        scratch_types=[
            pltpu.VMEM((4, wpw, W), jnp.int32),
            pltpu.VMEM((W, d), jnp.float32),
            pltpu.VMEM((W, d), jnp.float32),
            pltpu.VMEM((W, d), jnp.float32),
            pltpu.VMEM((W, d), jnp.float32),
            pltpu.SemaphoreType.DMA,
        ],
        compiler_params=pltpu.CompilerParams(use_tc_tiling_on_sc=False),
    )
    out = run(feature_miRNA, feature_gene, feature_drug, idx)
    return out.reshape(b, l, d)
